# Initial kernel scaffold; baseline (speedup 1.0000x reference)
import jax
import jax.numpy as jnp
from jax.experimental import pallas as pl


def kernel(node_feats, edge_feats, edge_index, node_graph_ids, params):
    raise NotImplementedError("write your pallas kernel here")


# speedup vs baseline: 1.0820x; 1.0820x over previous
import functools

import jax
import jax.numpy as jnp
from jax import lax
from jax.experimental import pallas as pl
from jax.experimental.pallas import tpu as pltpu
from jax.experimental.pallas import tpu_sc as plsc

N_NODES = 10000
N_EDGES = 320000
D_NODE = 128
D_EDGE = 16
HID = 128
N_GRAPHS = 64

F32 = jnp.float32
_P = lax.Precision.HIGHEST


def _mm(a, b):
    return lax.dot_general(a, b, (((1,), (0,)), ((), ())), precision=_P)


def _dense_prep_body(nf, lw, lb, rw, rb, left_out, right_out):
    x = nf[...]
    left_out[...] = _mm(x, lw[...]) + lb[...]
    right_out[...] = _mm(x, rw[...]) + rb[...]


def _dense_prep(node_feats, lw, lb, rw, rb):
    return pl.pallas_call(
        _dense_prep_body,
        out_shape=(
            jax.ShapeDtypeStruct((N_NODES, HID), F32),
            jax.ShapeDtypeStruct((N_NODES, HID), F32),
        ),
    )(node_feats, lw, lb, rw, rb)


_EBLK = 512


def _edge_chain_body(ef, gl, gr, we2e, be2e, we2n, be2n, wua, wub, bu,
                     w2e2n, b2e2n, out):
    e = ef[...]
    second = jnp.maximum(_mm(e, we2e[...]) + be2e[...], 0.0)
    e2n = jnp.maximum(_mm(e, we2n[...]) + be2n[...], 0.0)
    first = jnp.maximum(gl[...] + gr[...], 0.0)
    new_edge = jnp.maximum(_mm(first, wua[...]) + _mm(second, wub[...]) + bu[...], 0.0)
    z = jnp.maximum(_mm(new_edge, w2e2n[...]) + b2e2n[...], 0.0)
    out[...] = jnp.concatenate([e2n, z], axis=1)


def _edge_chain(edge_feats, gl, gr, we2e, be2e, we2n, be2n, wua, wub, bu,
                w2e2n, b2e2n):
    nblk = N_EDGES // _EBLK
    full = lambda a: pl.BlockSpec(a.shape, lambda i: (0, 0))
    return pl.pallas_call(
        _edge_chain_body,
        grid=(nblk,),
        in_specs=[
            pl.BlockSpec((_EBLK, D_EDGE), lambda i: (i, 0)),
            pl.BlockSpec((_EBLK, HID), lambda i: (i, 0)),
            pl.BlockSpec((_EBLK, HID), lambda i: (i, 0)),
            full(we2e), full(be2e), full(we2n), full(be2n), full(wua),
            full(wub), full(bu), full(w2e2n), full(b2e2n),
        ],
        out_specs=pl.BlockSpec((_EBLK, 2 * HID), lambda i: (i, 0)),
        out_shape=jax.ShapeDtypeStruct((N_EDGES, 2 * HID), F32),
    )(edge_feats, gl, gr, we2e, be2e, we2n, be2n, wua, wub, bu, w2e2n, b2e2n)


_GCH = 80


def _sc_gather(left, right, src, dst):
    info = plsc.get_sparse_core_info()
    nw = info.num_cores * info.num_subcores
    per_w = N_EDGES // nw
    nch = per_w // _GCH
    mesh = plsc.VectorSubcoreMesh(core_axis_name="c", subcore_axis_name="s")

    @functools.partial(
        pl.kernel, mesh=mesh,
        out_type=(jax.ShapeDtypeStruct((N_EDGES, HID), F32),
                  jax.ShapeDtypeStruct((N_EDGES, HID), F32)),
        scratch_types=[
            pltpu.VMEM((_GCH,), jnp.int32),
            pltpu.VMEM((_GCH,), jnp.int32),
            pltpu.VMEM((_GCH, HID), F32),
            pltpu.VMEM((_GCH, HID), F32),
            pltpu.SemaphoreType.DMA,
            pltpu.SemaphoreType.DMA,
        ])
    def gk(left_h, right_h, src_h, dst_h, outl_h, outr_h,
           srcv, dstv, bl, br, s1, s2):
        wid = lax.axis_index("s") * info.num_cores + lax.axis_index("c")
        base = wid * per_w

        def body(ci, carry):
            off = base + ci * _GCH
            pltpu.sync_copy(src_h.at[pl.ds(off, _GCH)], srcv)
            pltpu.sync_copy(dst_h.at[pl.ds(off, _GCH)], dstv)
            a = pltpu.async_copy(left_h.at[srcv], bl, s1)
            b = pltpu.async_copy(right_h.at[dstv], br, s2)
            a.wait()
            b.wait()
            pltpu.sync_copy(bl, outl_h.at[pl.ds(off, _GCH)])
            pltpu.sync_copy(br, outr_h.at[pl.ds(off, _GCH)])
            return carry

        lax.fori_loop(0, nch, body, 0)

    return gk(left, right, src, dst)


_SCH = 80


def _sc_scatter(scat, dst):
    info = plsc.get_sparse_core_info()
    ns = info.num_subcores
    per_t = N_EDGES // ns
    nch = per_t // _SCH
    _RA = 640
    _RL = N_NODES - (ns - 1) * _RA
    mesh = plsc.VectorSubcoreMesh(core_axis_name="c", subcore_axis_name="s",
                                  num_cores=1)

    @functools.partial(
        pl.kernel, mesh=mesh,
        out_type=jax.ShapeDtypeStruct((N_NODES, 2 * HID), F32),
        scratch_types=[
            pltpu.VMEM((_SCH,), jnp.int32),
            pltpu.VMEM((_SCH, HID), F32),
            pltpu.VMEM((_SCH, HID), F32),
            pltpu.VMEM_SHARED((N_NODES, HID), F32),
        ])
    def sk(scat_h, dst_h, out_h, dstv, buf, zbuf, acc):
        s = lax.axis_index("s")

        def zinit(i, carry):
            zbuf[i // 8, pl.ds((i % 8) * 16, 16)] = jnp.zeros((16,), F32)
            return carry

        lax.fori_loop(0, _SCH * 8, zinit, 0)
        for col0 in (0, HID):
            @pl.when(s < ns - 1)
            def _():
                for k in range(_RA // _SCH):
                    pltpu.sync_copy(
                        zbuf, acc.at[pl.ds(s * _RA + k * _SCH, _SCH)])

            @pl.when(s == ns - 1)
            def _():
                for k in range(_RL // _SCH):
                    pltpu.sync_copy(
                        zbuf,
                        acc.at[pl.ds((ns - 1) * _RA + k * _SCH, _SCH)])

            plsc.subcore_barrier()

            def body(ci, carry):
                off = s * per_t + ci * _SCH
                pltpu.sync_copy(dst_h.at[pl.ds(off, _SCH)], dstv)
                pltpu.sync_copy(
                    scat_h.at[pl.ds(off, _SCH), pl.ds(col0, HID)], buf)
                pltpu.sync_copy(buf, acc.at[dstv], add=True)
                return carry

            lax.fori_loop(0, nch, body, 0)
            plsc.subcore_barrier()

            @pl.when(s < ns - 1)
            def _():
                pltpu.sync_copy(
                    acc.at[pl.ds(s * _RA, _RA)],
                    out_h.at[pl.ds(s * _RA, _RA), pl.ds(col0, HID)])

            @pl.when(s == ns - 1)
            def _():
                pltpu.sync_copy(
                    acc.at[pl.ds((ns - 1) * _RA, _RL)],
                    out_h.at[pl.ds((ns - 1) * _RA, _RL), pl.ds(col0, HID)])

            plsc.subcore_barrier()

    return sk(scat, dst)


def _finalize_body(nf, en, en2, gid, wn2n, bn2n, wu1a, wu1b, bu1,
                   w2n2n, b2n2n, wu2a, wu2b, bu2, aw, awb,
                   p1w, p1b, p2w, p2b, out, hm_ref):
    act = lambda x: jnp.maximum(x, 0.0)
    node_node = act(_mm(nf[...], wn2n[...]) + bn2n[...])
    new_node = act(_mm(node_node, wu1a[...]) + _mm(en[...], wu1b[...]) + bu1[...])
    node_node2 = act(_mm(new_node, w2n2n[...]) + b2n2n[...])
    h = act(_mm(node_node2, wu2a[...]) + _mm(en2[...], wu2b[...]) + bu2[...])
    s = _mm(h, aw[...]) + awb[...]
    atom_w = 1.0 / (1.0 + jnp.exp(-s))
    weighted = h * atom_w
    g = gid[...]
    onehot = (g == lax.broadcasted_iota(jnp.int32, (1, N_GRAPHS), 1)
              ).astype(F32)
    hsum = lax.dot_general(onehot, weighted, (((0,), (0,)), ((), ())), precision=_P)

    def mbody(i, _):
        m = g == i
        col = jnp.where(m, h, -jnp.inf).max(axis=0, keepdims=True)
        hm_ref[pl.ds(i, 1), :] = col
        return 0

    lax.fori_loop(0, N_GRAPHS, mbody, 0)
    g_feats = jnp.concatenate([hsum, hm_ref[...]], axis=1)
    out[...] = _mm(_mm(g_feats, p1w[...]) + p1b[...], p2w[...]) + p2b[...]


def _finalize(nf, en, en2, gid, wn2n, bn2n, wu1a, wu1b, bu1, w2n2n, b2n2n,
              wu2a, wu2b, bu2, aw, awb, p1w, p1b, p2w, p2b):
    return pl.pallas_call(
        _finalize_body,
        out_shape=jax.ShapeDtypeStruct((N_GRAPHS, 1), F32),
        scratch_shapes=[pltpu.VMEM((N_GRAPHS, HID), F32)],
    )(nf, en, en2, gid, wn2n, bn2n, wu1a, wu1b, bu1, w2n2n, b2n2n,
      wu2a, wu2b, bu2, aw, awb, p1w, p1b, p2w, p2b)


def kernel(node_feats, edge_feats, edge_index, node_graph_ids, params):
    src = edge_index[0]
    dst = edge_index[1]
    p = params
    row = lambda v: v.reshape(1, -1)

    lw, lb = p['l1_left']
    rw, rb = p['l1_right']
    left, right = _dense_prep(node_feats, lw, row(lb), rw, row(rb))

    gl, gr = _sc_gather(left, right, src, dst)

    we2e, be2e = p['l1_e2e']
    we2n, be2n = p['l1_e2n']
    wu, bu = p['l1_upde']
    w2e2n, b2e2n = p['l2_e2n']
    scat = _edge_chain(edge_feats, gl, gr, we2e, row(be2e), we2n, row(be2n),
                       wu[:HID], wu[HID:], row(bu), w2e2n, row(b2e2n))

    acc = _sc_scatter(scat, dst)
    en = acc[:, :HID]
    en2 = acc[:, HID:]

    wn2n, bn2n = p['l1_n2n']
    wu1, bu1 = p['l1_updn']
    w2n2n, b2n2n = p['l2_n2n']
    wu2, bu2 = p['l2_updn']
    aw, awb = p['aw']
    p1w, p1b = p['pred1']
    p2w, p2b = p['pred2']
    return _finalize(node_feats, en, en2,
                     node_graph_ids.reshape(N_NODES, 1).astype(jnp.int32),
                     wn2n, row(bn2n), wu1[:HID], wu1[HID:], row(bu1),
                     w2n2n, row(b2n2n), wu2[:HID], wu2[HID:], row(bu2),
                     aw, row(awb), p1w, row(p1b), p2w, row(p2b))


# speedup vs baseline: 1.3596x; 1.2566x over previous
import functools

import jax
import jax.numpy as jnp
from jax import lax
from jax.experimental import pallas as pl
from jax.experimental.pallas import tpu as pltpu
from jax.experimental.pallas import tpu_sc as plsc

N_NODES = 10000
N_EDGES = 320000
D_NODE = 128
D_EDGE = 16
HID = 128
N_GRAPHS = 64

F32 = jnp.float32
_P = lax.Precision.HIGHEST


def _mm(a, b):
    return lax.dot_general(a, b, (((1,), (0,)), ((), ())), precision=_P)


def _dense_prep_body(nf, lw, lb, rw, rb, left_out, right_out):
    x = nf[...]
    left_out[...] = _mm(x, lw[...]) + lb[...]
    right_out[...] = _mm(x, rw[...]) + rb[...]


def _dense_prep(node_feats, lw, lb, rw, rb):
    return pl.pallas_call(
        _dense_prep_body,
        out_shape=(
            jax.ShapeDtypeStruct((N_NODES, HID), F32),
            jax.ShapeDtypeStruct((N_NODES, HID), F32),
        ),
    )(node_feats, lw, lb, rw, rb)


_EBLK = 512


def _edge_chain_body(ef, gl, gr, we2e, be2e, we2n, be2n, wua, wub, bu,
                     w2e2n, b2e2n, out):
    e = ef[...]
    second = jnp.maximum(_mm(e, we2e[...]) + be2e[...], 0.0)
    e2n = jnp.maximum(_mm(e, we2n[...]) + be2n[...], 0.0)
    first = jnp.maximum(gl[...] + gr[...], 0.0)
    new_edge = jnp.maximum(_mm(first, wua[...]) + _mm(second, wub[...]) + bu[...], 0.0)
    z = jnp.maximum(_mm(new_edge, w2e2n[...]) + b2e2n[...], 0.0)
    out[...] = jnp.concatenate([e2n, z], axis=1)


def _edge_chain(edge_feats, gl, gr, we2e, be2e, we2n, be2n, wua, wub, bu,
                w2e2n, b2e2n):
    nblk = N_EDGES // _EBLK
    full = lambda a: pl.BlockSpec(a.shape, lambda i: (0, 0))
    return pl.pallas_call(
        _edge_chain_body,
        grid=(nblk,),
        in_specs=[
            pl.BlockSpec((_EBLK, D_EDGE), lambda i: (i, 0)),
            pl.BlockSpec((_EBLK, HID), lambda i: (i, 0)),
            pl.BlockSpec((_EBLK, HID), lambda i: (i, 0)),
            full(we2e), full(be2e), full(we2n), full(be2n), full(wua),
            full(wub), full(bu), full(w2e2n), full(b2e2n),
        ],
        out_specs=pl.BlockSpec((_EBLK, 2 * HID), lambda i: (i, 0)),
        out_shape=jax.ShapeDtypeStruct((N_EDGES, 2 * HID), F32),
    )(edge_feats, gl, gr, we2e, be2e, we2n, be2n, wua, wub, bu, w2e2n, b2e2n)


_GCH = 80


def _sc_gather(left, right, src, dst):
    info = plsc.get_sparse_core_info()
    nc = info.num_cores
    nw = nc * info.num_subcores
    per_w = N_EDGES // nw
    nch = per_w // _GCH
    mesh = plsc.VectorSubcoreMesh(core_axis_name="c", subcore_axis_name="s")

    @functools.partial(
        pl.kernel, mesh=mesh,
        out_type=(jax.ShapeDtypeStruct((N_EDGES, HID), F32),
                  jax.ShapeDtypeStruct((N_EDGES, HID), F32)),
        scratch_types=[
            pltpu.VMEM((per_w,), jnp.int32),
            pltpu.VMEM((per_w,), jnp.int32),
        ] + [pltpu.VMEM((_GCH, HID), F32)] * 8
          + [pltpu.SemaphoreType.DMA] * 8)
    def gk(left_h, right_h, src_h, dst_h, outl_h, outr_h,
           srcv, dstv, bl0, br0, bl1, br1, bl2, br2, bl3, br3,
           g0, g1, g2, g3, w0, w1, w2, w3):
        wid = lax.axis_index("s") * nc + lax.axis_index("c")
        base = wid * per_w
        pltpu.sync_copy(src_h.at[pl.ds(base, per_w)], srcv)
        pltpu.sync_copy(dst_h.at[pl.ds(base, per_w)], dstv)
        slots = ((bl0, br0, g0, w0), (bl1, br1, g1, w1),
                 (bl2, br2, g2, w2), (bl3, br3, g3, w3))

        def issue(ci, k):
            bl, br, g, _ = slots[k]
            i = pl.ds(ci * _GCH, _GCH)
            pltpu.async_copy(left_h.at[srcv.at[i]], bl, g)
            pltpu.async_copy(right_h.at[dstv.at[i]], br, g)

        def wait_gather(k):
            bl, br, g, _ = slots[k]
            pltpu.make_async_copy(left_h.at[pl.ds(0, _GCH)], bl, g).wait()
            pltpu.make_async_copy(left_h.at[pl.ds(0, _GCH)], br, g).wait()

        def write(ci, k):
            bl, br, _, w = slots[k]
            o = pl.ds(base + ci * _GCH, _GCH)
            pltpu.async_copy(bl, outl_h.at[o], w)
            pltpu.async_copy(br, outr_h.at[o], w)

        def wait_write(k):
            bl, br, _, w = slots[k]
            pltpu.make_async_copy(bl, outl_h.at[pl.ds(0, _GCH)], w).wait()
            pltpu.make_async_copy(br, outl_h.at[pl.ds(0, _GCH)], w).wait()

        issue(0, 0)
        issue(1, 1)

        def body(q, carry):
            for k in range(4):
                ci = 4 * q + k
                wait_gather(k)
                write(ci, k)
                ci2 = ci + 2
                s2 = (k + 2) % 4

                @pl.when(ci >= 2)
                def _():
                    wait_write(s2)

                @pl.when(ci2 < nch)
                def _():
                    issue(ci2, s2)
            return carry

        lax.fori_loop(0, (nch - 1) // 4, body, 0)
        wait_gather(0)
        write(nch - 1, 0)
        for k in (2, 3, 0):
            wait_write(k)

    return gk(left, right, src, dst)


_SCH = 80


def _sc_scatter_half(scat, dst, col0):
    info = plsc.get_sparse_core_info()
    ns = info.num_subcores
    per_t = N_EDGES // ns
    nch = per_t // _SCH
    _RA = 640
    _RL = N_NODES - (ns - 1) * _RA
    mesh = plsc.VectorSubcoreMesh(core_axis_name="c", subcore_axis_name="s",
                                  num_cores=1)

    @functools.partial(
        pl.kernel, mesh=mesh,
        out_type=jax.ShapeDtypeStruct((N_NODES, HID), F32),
        scratch_types=[
            pltpu.VMEM((_SCH,), jnp.int32),
            pltpu.VMEM((_SCH,), jnp.int32),
            pltpu.VMEM((_SCH, HID), F32),
            pltpu.VMEM((_SCH, HID), F32),
            pltpu.VMEM((_SCH, HID), F32),
            pltpu.VMEM_SHARED((N_NODES, HID), F32),
        ] + [pltpu.SemaphoreType.DMA] * 4)
    def sk(scat_h, dst_h, out_h, di0, di1, b0, b1, zbuf, acc,
           l0, l1, a0, a1):
        s = lax.axis_index("s")

        def zinit(i, carry):
            zbuf[i // 8, pl.ds((i % 8) * 16, 16)] = jnp.zeros((16,), F32)
            return carry

        lax.fori_loop(0, _SCH * 8, zinit, 0)

        @pl.when(s < ns - 1)
        def _():
            for k in range(_RA // _SCH):
                pltpu.sync_copy(zbuf, acc.at[pl.ds(s * _RA + k * _SCH, _SCH)])

        @pl.when(s == ns - 1)
        def _():
            for k in range(_RL // _SCH):
                pltpu.sync_copy(
                    zbuf, acc.at[pl.ds((ns - 1) * _RA + k * _SCH, _SCH)])

        plsc.subcore_barrier()

        slots = ((di0, b0, l0, a0), (di1, b1, l1, a1))

        def load(ci, k):
            di, b, l, _ = slots[k]
            off = s * per_t + ci * _SCH
            pltpu.async_copy(dst_h.at[pl.ds(off, _SCH)], di, l)
            pltpu.async_copy(
                scat_h.at[pl.ds(off, _SCH), pl.ds(col0, HID)], b, l)

        def wait_load(k):
            di, b, l, _ = slots[k]
            pltpu.make_async_copy(dst_h.at[pl.ds(0, _SCH)], di, l).wait()
            pltpu.make_async_copy(
                scat_h.at[pl.ds(0, _SCH), pl.ds(0, HID)], b, l).wait()

        def add(k):
            di, b, _, a = slots[k]
            pltpu.async_copy(b, acc.at[di], a, add=True)

        def wait_add(k):
            di, b, _, a = slots[k]
            pltpu.make_async_copy(b, acc.at[pl.ds(0, _SCH)], a).wait()

        load(0, 0)
        load(1, 1)

        def body(p, carry):
            c = 2 * p
            wait_load(0)
            add(0)
            wait_load(1)
            add(1)

            @pl.when(c + 2 < nch)
            def _():
                wait_add(0)
                load(c + 2, 0)
                wait_add(1)
                load(c + 3, 1)
            return carry

        lax.fori_loop(0, nch // 2, body, 0)
        wait_add(0)
        wait_add(1)
        plsc.subcore_barrier()

        @pl.when(s < ns - 1)
        def _():
            pltpu.sync_copy(acc.at[pl.ds(s * _RA, _RA)],
                            out_h.at[pl.ds(s * _RA, _RA)])

        @pl.when(s == ns - 1)
        def _():
            pltpu.sync_copy(acc.at[pl.ds((ns - 1) * _RA, _RL)],
                            out_h.at[pl.ds((ns - 1) * _RA, _RL)])

    return sk(scat, dst)


def _finalize_body(nf, en, en2, gid, wn2n, bn2n, wu1a, wu1b, bu1,
                   w2n2n, b2n2n, wu2a, wu2b, bu2, aw, awb,
                   p1w, p1b, p2w, p2b, out, hm_ref):
    act = lambda x: jnp.maximum(x, 0.0)
    node_node = act(_mm(nf[...], wn2n[...]) + bn2n[...])
    new_node = act(_mm(node_node, wu1a[...]) + _mm(en[...], wu1b[...]) + bu1[...])
    node_node2 = act(_mm(new_node, w2n2n[...]) + b2n2n[...])
    h = act(_mm(node_node2, wu2a[...]) + _mm(en2[...], wu2b[...]) + bu2[...])
    s = _mm(h, aw[...]) + awb[...]
    atom_w = 1.0 / (1.0 + jnp.exp(-s))
    weighted = h * atom_w
    g = gid[...]
    onehot = (g == lax.broadcasted_iota(jnp.int32, (1, N_GRAPHS), 1)
              ).astype(F32)
    hsum = lax.dot_general(onehot, weighted, (((0,), (0,)), ((), ())), precision=_P)

    def mbody(i, _):
        m = g == i
        col = jnp.where(m, h, -jnp.inf).max(axis=0, keepdims=True)
        hm_ref[pl.ds(i, 1), :] = col
        return 0

    lax.fori_loop(0, N_GRAPHS, mbody, 0)
    g_feats = jnp.concatenate([hsum, hm_ref[...]], axis=1)
    out[...] = _mm(_mm(g_feats, p1w[...]) + p1b[...], p2w[...]) + p2b[...]


def _finalize(nf, en, en2, gid, wn2n, bn2n, wu1a, wu1b, bu1, w2n2n, b2n2n,
              wu2a, wu2b, bu2, aw, awb, p1w, p1b, p2w, p2b):
    return pl.pallas_call(
        _finalize_body,
        out_shape=jax.ShapeDtypeStruct((N_GRAPHS, 1), F32),
        scratch_shapes=[pltpu.VMEM((N_GRAPHS, HID), F32)],
    )(nf, en, en2, gid, wn2n, bn2n, wu1a, wu1b, bu1, w2n2n, b2n2n,
      wu2a, wu2b, bu2, aw, awb, p1w, p1b, p2w, p2b)


def kernel(node_feats, edge_feats, edge_index, node_graph_ids, params):
    src = edge_index[0]
    dst = edge_index[1]
    p = params
    row = lambda v: v.reshape(1, -1)

    lw, lb = p['l1_left']
    rw, rb = p['l1_right']
    left, right = _dense_prep(node_feats, lw, row(lb), rw, row(rb))

    gl, gr = _sc_gather(left, right, src, dst)

    we2e, be2e = p['l1_e2e']
    we2n, be2n = p['l1_e2n']
    wu, bu = p['l1_upde']
    w2e2n, b2e2n = p['l2_e2n']
    scat = _edge_chain(edge_feats, gl, gr, we2e, row(be2e), we2n, row(be2n),
                       wu[:HID], wu[HID:], row(bu), w2e2n, row(b2e2n))

    en = _sc_scatter_half(scat, dst, 0)
    en2 = _sc_scatter_half(scat, dst, HID)

    wn2n, bn2n = p['l1_n2n']
    wu1, bu1 = p['l1_updn']
    w2n2n, b2n2n = p['l2_n2n']
    wu2, bu2 = p['l2_updn']
    aw, awb = p['aw']
    p1w, p1b = p['pred1']
    p2w, p2b = p['pred2']
    return _finalize(node_feats, en, en2,
                     node_graph_ids.reshape(N_NODES, 1).astype(jnp.int32),
                     wn2n, row(bn2n), wu1[:HID], wu1[HID:], row(bu1),
                     w2n2n, row(b2n2n), wu2[:HID], wu2[HID:], row(bu2),
                     aw, row(awb), p1w, row(p1b), p2w, row(p2b))


# speedup vs baseline: 1.6173x; 1.1895x over previous
import functools

import jax
import jax.numpy as jnp
from jax import lax
from jax.experimental import pallas as pl
from jax.experimental.pallas import tpu as pltpu
from jax.experimental.pallas import tpu_sc as plsc

N_NODES = 10000
N_EDGES = 320000
D_NODE = 128
D_EDGE = 16
HID = 128
N_GRAPHS = 64

F32 = jnp.float32
_P = lax.Precision.HIGHEST


def _mm(a, b):
    return lax.dot_general(a, b, (((1,), (0,)), ((), ())), precision=_P)


def _mmh(a, b):
    bf = jnp.bfloat16
    dn = (((1,), (0,)), ((), ()))
    a_hi = a.astype(bf)
    b_hi = b.astype(bf)
    a_lo = (a - a_hi.astype(F32)).astype(bf)
    b_lo = (b - b_hi.astype(F32)).astype(bf)
    d = lambda x, y: lax.dot_general(x, y, dn, preferred_element_type=F32)
    return d(a_hi, b_hi) + d(a_lo, b_hi) + d(a_hi, b_lo)


def _dense_prep_body(nf, lw, lb, rw, rb, left_out, right_out):
    x = nf[...]
    left_out[...] = _mm(x, lw[...]) + lb[...]
    right_out[...] = _mm(x, rw[...]) + rb[...]


def _dense_prep(node_feats, lw, lb, rw, rb):
    return pl.pallas_call(
        _dense_prep_body,
        out_shape=(
            jax.ShapeDtypeStruct((N_NODES, HID), F32),
            jax.ShapeDtypeStruct((N_NODES, HID), F32),
        ),
    )(node_feats, lw, lb, rw, rb)


_EBLK = 512


def _edge_chain_body(ef, gl, gr, we2e, be2e, we2n, be2n, wua, wub, bu,
                     w2e2n, b2e2n, out):
    e = ef[...]
    second = jnp.maximum(_mmh(e, we2e[...]) + be2e[...], 0.0)
    e2n = jnp.maximum(_mmh(e, we2n[...]) + be2n[...], 0.0)
    first = jnp.maximum(gl[...] + gr[...], 0.0)
    new_edge = jnp.maximum(_mmh(first, wua[...]) + _mmh(second, wub[...]) + bu[...], 0.0)
    z = jnp.maximum(_mmh(new_edge, w2e2n[...]) + b2e2n[...], 0.0)
    out[...] = jnp.concatenate([e2n, z], axis=1)


def _edge_chain(edge_feats, gl, gr, we2e, be2e, we2n, be2n, wua, wub, bu,
                w2e2n, b2e2n):
    nblk = N_EDGES // _EBLK
    full = lambda a: pl.BlockSpec(a.shape, lambda i: (0, 0))
    return pl.pallas_call(
        _edge_chain_body,
        grid=(nblk,),
        in_specs=[
            pl.BlockSpec((_EBLK, D_EDGE), lambda i: (i, 0)),
            pl.BlockSpec((_EBLK, HID), lambda i: (i, 0)),
            pl.BlockSpec((_EBLK, HID), lambda i: (i, 0)),
            full(we2e), full(be2e), full(we2n), full(be2n), full(wua),
            full(wub), full(bu), full(w2e2n), full(b2e2n),
        ],
        out_specs=pl.BlockSpec((_EBLK, 2 * HID), lambda i: (i, 0)),
        out_shape=jax.ShapeDtypeStruct((N_EDGES, 2 * HID), F32),
    )(edge_feats, gl, gr, we2e, be2e, we2n, be2n, wua, wub, bu, w2e2n, b2e2n)


_GCH = 80


def _sc_gather(left, right, src, dst):
    info = plsc.get_sparse_core_info()
    nc = info.num_cores
    nw = nc * info.num_subcores
    per_w = N_EDGES // nw
    nch = per_w // _GCH
    mesh = plsc.VectorSubcoreMesh(core_axis_name="c", subcore_axis_name="s")

    @functools.partial(
        pl.kernel, mesh=mesh,
        out_type=(jax.ShapeDtypeStruct((N_EDGES, HID), F32),
                  jax.ShapeDtypeStruct((N_EDGES, HID), F32)),
        scratch_types=[
            pltpu.VMEM((per_w,), jnp.int32),
            pltpu.VMEM((per_w,), jnp.int32),
        ] + [pltpu.VMEM((_GCH, HID), F32)] * 8
          + [pltpu.SemaphoreType.DMA] * 8)
    def gk(left_h, right_h, src_h, dst_h, outl_h, outr_h,
           srcv, dstv, bl0, br0, bl1, br1, bl2, br2, bl3, br3,
           g0, g1, g2, g3, w0, w1, w2, w3):
        wid = lax.axis_index("s") * nc + lax.axis_index("c")
        base = wid * per_w
        pltpu.sync_copy(src_h.at[pl.ds(base, per_w)], srcv)
        pltpu.sync_copy(dst_h.at[pl.ds(base, per_w)], dstv)
        slots = ((bl0, br0, g0, w0), (bl1, br1, g1, w1),
                 (bl2, br2, g2, w2), (bl3, br3, g3, w3))

        def issue(ci, k):
            bl, br, g, _ = slots[k]
            i = pl.ds(ci * _GCH, _GCH)
            pltpu.async_copy(left_h.at[srcv.at[i]], bl, g)
            pltpu.async_copy(right_h.at[dstv.at[i]], br, g)

        def wait_gather(k):
            bl, br, g, _ = slots[k]
            pltpu.make_async_copy(left_h.at[pl.ds(0, _GCH)], bl, g).wait()
            pltpu.make_async_copy(left_h.at[pl.ds(0, _GCH)], br, g).wait()

        def write(ci, k):
            bl, br, _, w = slots[k]
            o = pl.ds(base + ci * _GCH, _GCH)
            pltpu.async_copy(bl, outl_h.at[o], w)
            pltpu.async_copy(br, outr_h.at[o], w)

        def wait_write(k):
            bl, br, _, w = slots[k]
            pltpu.make_async_copy(bl, outl_h.at[pl.ds(0, _GCH)], w).wait()
            pltpu.make_async_copy(br, outl_h.at[pl.ds(0, _GCH)], w).wait()

        issue(0, 0)
        issue(1, 1)

        def body(q, carry):
            for k in range(4):
                ci = 4 * q + k
                wait_gather(k)
                write(ci, k)
                ci2 = ci + 2
                s2 = (k + 2) % 4

                @pl.when(ci >= 2)
                def _():
                    wait_write(s2)

                @pl.when(ci2 < nch)
                def _():
                    issue(ci2, s2)
            return carry

        lax.fori_loop(0, (nch - 1) // 4, body, 0)
        wait_gather(0)
        write(nch - 1, 0)
        for k in (2, 3, 0):
            wait_write(k)

    return gk(left, right, src, dst)


_SCH = 80


def _sc_scatter_half(scat, dst, col0):
    info = plsc.get_sparse_core_info()
    ns = info.num_subcores
    per_t = N_EDGES // ns
    nch = per_t // _SCH
    _RA = 640
    _RL = N_NODES - (ns - 1) * _RA
    mesh = plsc.VectorSubcoreMesh(core_axis_name="c", subcore_axis_name="s",
                                  num_cores=1)

    @functools.partial(
        pl.kernel, mesh=mesh,
        out_type=jax.ShapeDtypeStruct((N_NODES, HID), F32),
        scratch_types=[
            pltpu.VMEM((_SCH,), jnp.int32),
            pltpu.VMEM((_SCH,), jnp.int32),
            pltpu.VMEM((_SCH, HID), F32),
            pltpu.VMEM((_SCH, HID), F32),
            pltpu.VMEM((_SCH, HID), F32),
            pltpu.VMEM_SHARED((N_NODES, HID), F32),
        ] + [pltpu.SemaphoreType.DMA] * 4)
    def sk(scat_h, dst_h, out_h, di0, di1, b0, b1, zbuf, acc,
           l0, l1, a0, a1):
        s = lax.axis_index("s")

        def zinit(i, carry):
            zbuf[i // 8, pl.ds((i % 8) * 16, 16)] = jnp.zeros((16,), F32)
            return carry

        lax.fori_loop(0, _SCH * 8, zinit, 0)

        @pl.when(s < ns - 1)
        def _():
            for k in range(_RA // _SCH):
                pltpu.sync_copy(zbuf, acc.at[pl.ds(s * _RA + k * _SCH, _SCH)])

        @pl.when(s == ns - 1)
        def _():
            for k in range(_RL // _SCH):
                pltpu.sync_copy(
                    zbuf, acc.at[pl.ds((ns - 1) * _RA + k * _SCH, _SCH)])

        plsc.subcore_barrier()

        slots = ((di0, b0, l0, a0), (di1, b1, l1, a1))

        def load(ci, k):
            di, b, l, _ = slots[k]
            off = s * per_t + ci * _SCH
            pltpu.async_copy(dst_h.at[pl.ds(off, _SCH)], di, l)
            pltpu.async_copy(
                scat_h.at[pl.ds(off, _SCH), pl.ds(col0, HID)], b, l)

        def wait_load(k):
            di, b, l, _ = slots[k]
            pltpu.make_async_copy(dst_h.at[pl.ds(0, _SCH)], di, l).wait()
            pltpu.make_async_copy(
                scat_h.at[pl.ds(0, _SCH), pl.ds(0, HID)], b, l).wait()

        def add(k):
            di, b, _, a = slots[k]
            pltpu.async_copy(b, acc.at[di], a, add=True)

        def wait_add(k):
            di, b, _, a = slots[k]
            pltpu.make_async_copy(b, acc.at[pl.ds(0, _SCH)], a).wait()

        load(0, 0)
        load(1, 1)

        def body(p, carry):
            c = 2 * p
            wait_load(0)
            add(0)
            wait_load(1)
            add(1)

            @pl.when(c + 2 < nch)
            def _():
                wait_add(0)
                load(c + 2, 0)
                wait_add(1)
                load(c + 3, 1)
            return carry

        lax.fori_loop(0, nch // 2, body, 0)
        wait_add(0)
        wait_add(1)
        plsc.subcore_barrier()

        @pl.when(s < ns - 1)
        def _():
            pltpu.sync_copy(acc.at[pl.ds(s * _RA, _RA)],
                            out_h.at[pl.ds(s * _RA, _RA)])

        @pl.when(s == ns - 1)
        def _():
            pltpu.sync_copy(acc.at[pl.ds((ns - 1) * _RA, _RL)],
                            out_h.at[pl.ds((ns - 1) * _RA, _RL)])

    return sk(scat, dst)


def _finalize_body(nf, en, en2, gid, wn2n, bn2n, wu1a, wu1b, bu1,
                   w2n2n, b2n2n, wu2a, wu2b, bu2, aw, awb,
                   p1w, p1b, p2w, p2b, out, hm_ref):
    act = lambda x: jnp.maximum(x, 0.0)
    node_node = act(_mm(nf[...], wn2n[...]) + bn2n[...])
    new_node = act(_mm(node_node, wu1a[...]) + _mm(en[...], wu1b[...]) + bu1[...])
    node_node2 = act(_mm(new_node, w2n2n[...]) + b2n2n[...])
    h = act(_mm(node_node2, wu2a[...]) + _mm(en2[...], wu2b[...]) + bu2[...])
    s = _mm(h, aw[...]) + awb[...]
    atom_w = 1.0 / (1.0 + jnp.exp(-s))
    weighted = h * atom_w
    g = gid[...]
    onehot = (g == lax.broadcasted_iota(jnp.int32, (1, N_GRAPHS), 1)
              ).astype(F32)
    hsum = lax.dot_general(onehot, weighted, (((0,), (0,)), ((), ())), precision=_P)

    def mbody(i, _):
        m = g == i
        col = jnp.where(m, h, -jnp.inf).max(axis=0, keepdims=True)
        hm_ref[pl.ds(i, 1), :] = col
        return 0

    lax.fori_loop(0, N_GRAPHS, mbody, 0)
    g_feats = jnp.concatenate([hsum, hm_ref[...]], axis=1)
    out[...] = _mm(_mm(g_feats, p1w[...]) + p1b[...], p2w[...]) + p2b[...]


def _finalize(nf, en, en2, gid, wn2n, bn2n, wu1a, wu1b, bu1, w2n2n, b2n2n,
              wu2a, wu2b, bu2, aw, awb, p1w, p1b, p2w, p2b):
    return pl.pallas_call(
        _finalize_body,
        out_shape=jax.ShapeDtypeStruct((N_GRAPHS, 1), F32),
        scratch_shapes=[pltpu.VMEM((N_GRAPHS, HID), F32)],
    )(nf, en, en2, gid, wn2n, bn2n, wu1a, wu1b, bu1, w2n2n, b2n2n,
      wu2a, wu2b, bu2, aw, awb, p1w, p1b, p2w, p2b)


def kernel(node_feats, edge_feats, edge_index, node_graph_ids, params):
    src = edge_index[0]
    dst = edge_index[1]
    p = params
    row = lambda v: v.reshape(1, -1)

    lw, lb = p['l1_left']
    rw, rb = p['l1_right']
    left, right = _dense_prep(node_feats, lw, row(lb), rw, row(rb))

    gl, gr = _sc_gather(left, right, src, dst)

    we2e, be2e = p['l1_e2e']
    we2n, be2n = p['l1_e2n']
    wu, bu = p['l1_upde']
    w2e2n, b2e2n = p['l2_e2n']
    scat = _edge_chain(edge_feats, gl, gr, we2e, row(be2e), we2n, row(be2n),
                       wu[:HID], wu[HID:], row(bu), w2e2n, row(b2e2n))

    en = _sc_scatter_half(scat, dst, 0)
    en2 = _sc_scatter_half(scat, dst, HID)

    wn2n, bn2n = p['l1_n2n']
    wu1, bu1 = p['l1_updn']
    w2n2n, b2n2n = p['l2_n2n']
    wu2, bu2 = p['l2_updn']
    aw, awb = p['aw']
    p1w, p1b = p['pred1']
    p2w, p2b = p['pred2']
    return _finalize(node_feats, en, en2,
                     node_graph_ids.reshape(N_NODES, 1).astype(jnp.int32),
                     wn2n, row(bn2n), wu1[:HID], wu1[HID:], row(bu1),
                     w2n2n, row(b2n2n), wu2[:HID], wu2[HID:], row(bu2),
                     aw, row(awb), p1w, row(p1b), p2w, row(p2b))
